# baseline (speedup 1.0000x reference)
import functools

import jax
import jax.numpy as jnp
from jax import lax
from jax.experimental import pallas as pl
from jax.experimental.pallas import tpu as pltpu
from jax.experimental.pallas import tpu_sc as plsc

D = 64
N_HOUR = 24
N_MIN = 4
N_TIME = N_HOUR * N_MIN
N_WDAY = 7
N_ROWS = N_TIME * N_WDAY

NUM_CORES = 2
NUM_SUBCORES = 16
NW = NUM_CORES * NUM_SUBCORES

CHUNK = 512
GATHER = 128
LANES = 16


def _table_body(h_ref, m_ref, w_ref, o_ref):
    r = lax.broadcasted_iota(jnp.int32, (N_ROWS, 1), 0)
    t = r // N_WDAY
    wd = r % N_WDAY
    h = t // N_MIN
    mn = t % N_MIN
    oh_h = (h == lax.broadcasted_iota(jnp.int32, (N_ROWS, N_HOUR), 1)).astype(
        jnp.float32
    )
    oh_m = (mn == lax.broadcasted_iota(jnp.int32, (N_ROWS, N_MIN), 1)).astype(
        jnp.float32
    )
    oh_w = (wd == lax.broadcasted_iota(jnp.int32, (N_ROWS, N_WDAY), 1)).astype(
        jnp.float32
    )
    o_ref[...] = (
        jnp.dot(oh_h, h_ref[...], preferred_element_type=jnp.float32)
        + jnp.dot(oh_m, m_ref[...], preferred_element_type=jnp.float32)
        + jnp.dot(oh_w, w_ref[...], preferred_element_type=jnp.float32)
    )


def _build_table(minute_embed, hour_embed, weekday_embed, interpret=False):
    return pl.pallas_call(
        _table_body,
        out_shape=jax.ShapeDtypeStruct((N_ROWS, D), jnp.float32),
        interpret=interpret,
    )(hour_embed, minute_embed, weekday_embed)


IDX_BLK = 6400


def _sc_gather(time_flat, weekday_flat, table):
    n = time_flat.shape[0]
    n_per_w = n // NW
    n_outer = n_per_w // CHUNK
    n_idx_blk = n_per_w // IDX_BLK
    mesh = plsc.VectorSubcoreMesh(core_axis_name="c", subcore_axis_name="s")

    @functools.partial(
        pl.kernel,
        mesh=mesh,
        compiler_params=pltpu.CompilerParams(use_tc_tiling_on_sc=False),
        out_type=jax.ShapeDtypeStruct((n, D), jnp.float32),
        scratch_types=[
            pltpu.VMEM((IDX_BLK,), jnp.int32),
            pltpu.VMEM((IDX_BLK,), jnp.int32),
            pltpu.VMEM((n_per_w,), jnp.int32),
            pltpu.VMEM((CHUNK, D), jnp.float32),
            pltpu.VMEM((CHUNK, D), jnp.float32),
            pltpu.SemaphoreType.DMA,
            pltpu.SemaphoreType.DMA,
            pltpu.SemaphoreType.DMA,
        ],
    )
    def body(
        time_hbm,
        wday_hbm,
        table_hbm,
        out_hbm,
        t_v,
        w_v,
        c_v,
        rows0,
        rows1,
        sem_g,
        sem_w0,
        sem_w1,
    ):
        wid = lax.axis_index("s") * NUM_CORES + lax.axis_index("c")
        base = wid * n_per_w

        def idx_block(k, carry):
            boff = k * IDX_BLK
            pltpu.sync_copy(time_hbm.at[pl.ds(base + boff, IDX_BLK)], t_v)
            pltpu.sync_copy(wday_hbm.at[pl.ds(base + boff, IDX_BLK)], w_v)

            def compute(j, carry2):
                sl = pl.ds(j * LANES, LANES)
                t = t_v[sl]
                w = w_v[sl]
                h = jnp.clip(t >> 2, 0, N_HOUR - 1)
                mn = t & 3
                wd = jnp.clip(w, 0, N_WDAY - 1)
                c_v[pl.ds(boff + j * LANES, LANES)] = (
                    h * (N_MIN * N_WDAY) + mn * N_WDAY + wd
                )
                return carry2

            lax.fori_loop(0, IDX_BLK // LANES, compute, 0)
            return carry

        lax.fori_loop(0, n_idx_blk, idx_block, 0)

        rows = (rows0, rows1)
        sem_w = (sem_w0, sem_w1)

        def outer(g, carry):
            for b in range(2):
                i = g * 2 + b
                off = base + i * CHUNK

                @pl.when(g > 0)
                def _wait_prev_write():
                    pltpu.make_async_copy(
                        rows[b], out_hbm.at[pl.ds(off, CHUNK)], sem_w[b]
                    ).wait()

                copies = []
                for s in range(CHUNK // GATHER):
                    csl = pl.ds(i * CHUNK + s * GATHER, GATHER)
                    rsl = pl.ds(s * GATHER, GATHER)
                    copies.append(
                        pltpu.async_copy(
                            table_hbm.at[c_v.at[csl]], rows[b].at[rsl], sem_g
                        )
                    )
                for cp in copies:
                    cp.wait()
                pltpu.async_copy(rows[b], out_hbm.at[pl.ds(off, CHUNK)], sem_w[b])
            return carry

        lax.fori_loop(0, n_outer // 2, outer, 0)
        for b in range(2):
            pltpu.make_async_copy(
                rows[b], out_hbm.at[pl.ds(base, CHUNK)], sem_w[b]
            ).wait()

    return body(time_flat, weekday_flat, table)


def kernel(time, weekday, minute_embed, hour_embed, weekday_embed):
    s, b = time.shape
    table = _build_table(minute_embed, hour_embed, weekday_embed)
    tf = time.reshape(-1).astype(jnp.int32)
    wf = weekday.reshape(-1).astype(jnp.int32)
    out = _sc_gather(tf, wf, table)
    return out.reshape(s, b, D)


# speedup vs baseline: 1.0351x; 1.0351x over previous
import functools

import jax
import jax.numpy as jnp
from jax import lax
from jax.experimental import pallas as pl
from jax.experimental.pallas import tpu as pltpu
from jax.experimental.pallas import tpu_sc as plsc

D = 64
N_HOUR = 24
N_MIN = 4
N_TIME = N_HOUR * N_MIN
N_WDAY = 7
N_ROWS = N_TIME * N_WDAY

NUM_CORES = 2
NUM_SUBCORES = 16
NW = NUM_CORES * NUM_SUBCORES

CHUNK = 256
LANES = 16


def _table_body(h_ref, m_ref, w_ref, o_ref):
    r = lax.broadcasted_iota(jnp.int32, (N_ROWS, 1), 0)
    t = r // N_WDAY
    wd = r % N_WDAY
    h = t // N_MIN
    mn = t % N_MIN
    oh_h = (h == lax.broadcasted_iota(jnp.int32, (N_ROWS, N_HOUR), 1)).astype(
        jnp.float32
    )
    oh_m = (mn == lax.broadcasted_iota(jnp.int32, (N_ROWS, N_MIN), 1)).astype(
        jnp.float32
    )
    oh_w = (wd == lax.broadcasted_iota(jnp.int32, (N_ROWS, N_WDAY), 1)).astype(
        jnp.float32
    )
    o_ref[...] = (
        jnp.dot(oh_h, h_ref[...], preferred_element_type=jnp.float32)
        + jnp.dot(oh_m, m_ref[...], preferred_element_type=jnp.float32)
        + jnp.dot(oh_w, w_ref[...], preferred_element_type=jnp.float32)
    )


def _build_table(minute_embed, hour_embed, weekday_embed, interpret=False):
    return pl.pallas_call(
        _table_body,
        out_shape=jax.ShapeDtypeStruct((N_ROWS, D), jnp.float32),
        interpret=interpret,
    )(hour_embed, minute_embed, weekday_embed)


def _sc_gather(time_flat, weekday_flat, table):
    n = time_flat.shape[0]
    n_per_w = n // NW
    n_outer = n_per_w // CHUNK
    mesh = plsc.VectorSubcoreMesh(core_axis_name="c", subcore_axis_name="s")

    @functools.partial(
        pl.kernel,
        mesh=mesh,
        compiler_params=pltpu.CompilerParams(needs_layout_passes=False),
        out_type=jax.ShapeDtypeStruct((n, D), jnp.float32),
        scratch_types=[
            pltpu.VMEM((N_ROWS * D,), jnp.float32),
            pltpu.VMEM((CHUNK,), jnp.int32),
            pltpu.VMEM((CHUNK,), jnp.int32),
            pltpu.VMEM((CHUNK,), jnp.int32),
            pltpu.VMEM((CHUNK, D), jnp.float32),
            pltpu.VMEM((CHUNK, D), jnp.float32),
            pltpu.SemaphoreType.DMA,
            pltpu.SemaphoreType.DMA,
            pltpu.SemaphoreType.DMA,
        ],
    )
    def body(
        time_hbm,
        wday_hbm,
        table_hbm,
        out_hbm,
        table_v,
        t_v,
        w_v,
        c_v,
        rows0,
        rows1,
        sem,
        sem_w0,
        sem_w1,
    ):
        wid = lax.axis_index("s") * NUM_CORES + lax.axis_index("c")
        base = wid * n_per_w

        pltpu.sync_copy(table_hbm, table_v)

        rows = (rows0, rows1)
        sem_w = (sem_w0, sem_w1)

        def outer(g, carry):
            for b in range(2):
                i = g * 2 + b
                off = base + i * CHUNK

                pltpu.sync_copy(time_hbm.at[pl.ds(off, CHUNK)], t_v)
                pltpu.sync_copy(wday_hbm.at[pl.ds(off, CHUNK)], w_v)

                def compute(j, carry2):
                    sl = pl.ds(j * LANES, LANES)
                    t = t_v[sl]
                    w = w_v[sl]
                    h = jnp.clip(t >> 2, 0, N_HOUR - 1)
                    mn = t & 3
                    wd = jnp.clip(w, 0, N_WDAY - 1)
                    c_v[sl] = (h * (N_MIN * N_WDAY) + mn * N_WDAY + wd) * D
                    return carry2

                lax.fori_loop(0, CHUNK // LANES, compute, 0)

                @pl.when(g > 0)
                def _wait_prev_write():
                    pltpu.make_async_copy(
                        rows[b], out_hbm.at[pl.ds(off, CHUNK)], sem_w[b]
                    ).wait()

                iota = jnp.arange(LANES, dtype=jnp.int32)

                def copy_rows(q, carry3):
                    for u in range(4):
                        k = q * 4 + u
                        kvec = jnp.full((LANES,), k, dtype=jnp.int32)
                        cb = plsc.load_gather(c_v, [kvec])
                        for v in range(D // LANES):
                            rows[b][k, pl.ds(v * LANES, LANES)] = plsc.load_gather(
                                table_v, [cb + (v * LANES) + iota]
                            )
                    return carry3

                lax.fori_loop(0, CHUNK // 4, copy_rows, 0)

                pltpu.async_copy(rows[b], out_hbm.at[pl.ds(off, CHUNK)], sem_w[b])
            return carry

        lax.fori_loop(0, n_outer // 2, outer, 0)
        for b in range(2):
            pltpu.make_async_copy(
                rows[b], out_hbm.at[pl.ds(base, CHUNK)], sem_w[b]
            ).wait()

    return body(time_flat, weekday_flat, table)


def kernel(time, weekday, minute_embed, hour_embed, weekday_embed):
    s, b = time.shape
    table = _build_table(minute_embed, hour_embed, weekday_embed).reshape(-1)
    tf = time.reshape(-1).astype(jnp.int32)
    wf = weekday.reshape(-1).astype(jnp.int32)
    out = _sc_gather(tf, wf, table)
    return out.reshape(s, b, D)
